# Initial kernel scaffold; baseline (speedup 1.0000x reference)
import jax
import jax.numpy as jnp
from jax.experimental import pallas as pl


def kernel(emb, edge_index):
    raise NotImplementedError("write your pallas kernel here")


# speedup vs baseline: 7.4253x; 7.4253x over previous
import functools

import jax
import jax.numpy as jnp
from jax import lax
from jax.experimental import pallas as pl
from jax.experimental.pallas import tpu as pltpu
from jax.experimental.pallas import tpu_sc as plsc

N = 50000
D = 64
H = 32
E = 800000
NLAYERS = 3

NC, NS, L = 2, 16, 16

CHUNK = 128
GROUP = 4
CPT = 392
GPT = CPT // GROUP
EPAD = NS * CPT * CHUNK
NCHUNKS = EPAD // CHUNK

RPT = 3136
NPAD = NS * RPT
RCHUNK = 64
NRC = RPT // RCHUNK
DUMMY = N


def _rsqrt16(d):
    i = lax.bitcast_convert_type(d, jnp.int32)
    i = jnp.int32(0x5F3759DF) - lax.shift_right_arithmetic(i, 1)
    y = lax.bitcast_convert_type(i, jnp.float32)
    for _ in range(3):
        y = y * (1.5 - 0.5 * d * y * y)
    return y


def _body(embs, srcp, dstp, out, y0, y1, acc, deg,
          srcb, dstb, rows, ab2, ob2, yb2, zb2, db, onesb, zb1, gsem):
    c = lax.axis_index("c")
    s = lax.axis_index("s")
    row_base = s * RPT
    chunk_base = s * CPT

    zero16 = jnp.zeros((L,), jnp.float32)
    one16 = jnp.ones((L,), jnp.float32)
    for r in range(RCHUNK):
        for h in range(H // L):
            zb2[r, pl.ds(h * L, L)] = zero16
    for v in range(RCHUNK // L):
        zb1[pl.ds(v * L, L)] = zero16
    for v in range(CHUNK // L):
        onesb[pl.ds(v * L, L)] = one16

    def _zero_slices(i, carry):
        base = row_base + i * RCHUNK
        pltpu.sync_copy(zb1, deg.at[pl.ds(base, RCHUNK)])
        pltpu.sync_copy(zb2, acc.at[pl.ds(base, RCHUNK)])
        return carry
    lax.fori_loop(0, NRC, _zero_slices, 0)
    plsc.subcore_barrier()

    def _deg_group(g, carry):
        cb = chunk_base + g * GROUP
        pltpu.sync_copy(dstp.at[pl.ds(cb, GROUP)], dstb)
        for j in range(GROUP):
            pltpu.sync_copy(onesb, deg.at[dstb.at[j]], add=True)
        return carry
    lax.fori_loop(0, GPT, _deg_group, 0)
    plsc.subcore_barrier()

    def _init_chunk(i, carry):
        base = row_base + i * RCHUNK
        pltpu.sync_copy(deg.at[pl.ds(base, RCHUNK)], db)
        pltpu.sync_copy(embs.at[c].at[pl.ds(base, RCHUNK)], ab2)

        def _rows(v, carry2):
            d = jnp.maximum(db[pl.ds(v * L, L)], 1.0)
            sdvec = _rsqrt16(d)
            for j in range(L):
                r = v * L + j
                sd = sdvec[j]
                for h in range(H // L):
                    e = ab2[r, pl.ds(h * L, L)]
                    yb2[r, pl.ds(h * L, L)] = e * sd
            return carry2
        lax.fori_loop(0, RCHUNK // L, _rows, 0)
        pltpu.sync_copy(ab2, out.at[c].at[pl.ds(base, RCHUNK)])
        pltpu.sync_copy(yb2, y0.at[c].at[pl.ds(base, RCHUNK)])
        return carry
    lax.fori_loop(0, NRC, _init_chunk, 0)
    plsc.subcore_barrier()

    ybufs = [y0, y1]
    for layer in range(NLAYERS):
        ycur = ybufs[layer % 2]
        ynext = ybufs[(layer + 1) % 2]

        def _agg_group(g, carry):
            cb = chunk_base + g * GROUP
            pltpu.sync_copy(srcp.at[pl.ds(cb, GROUP)], srcb)
            pltpu.sync_copy(dstp.at[pl.ds(cb, GROUP)], dstb)
            descs = [
                pltpu.async_copy(ycur.at[c].at[srcb.at[j]],
                                 rows.at[pl.ds(j * CHUNK, CHUNK)], gsem)
                for j in range(GROUP)
            ]
            for dsc in descs:
                dsc.wait()
            for j in range(GROUP):
                pltpu.sync_copy(rows.at[pl.ds(j * CHUNK, CHUNK)],
                                acc.at[dstb.at[j]], add=True)
            return carry
        lax.fori_loop(0, GPT, _agg_group, 0)
        plsc.subcore_barrier()

        last = layer == NLAYERS - 1

        def _post_chunk(i, carry):
            base = row_base + i * RCHUNK
            pltpu.sync_copy(deg.at[pl.ds(base, RCHUNK)], db)
            pltpu.sync_copy(acc.at[pl.ds(base, RCHUNK)], ab2)
            pltpu.sync_copy(out.at[c].at[pl.ds(base, RCHUNK)], ob2)

            def _rows(v, carry2):
                d = jnp.maximum(db[pl.ds(v * L, L)], 1.0)
                sdvec = _rsqrt16(d)
                for j in range(L):
                    r = v * L + j
                    sd = sdvec[j]
                    for h in range(H // L):
                        sl = pl.ds(h * L, L)
                        t = ab2[r, sl] * sd
                        o = ob2[r, sl] + t
                        if last:
                            ob2[r, sl] = o * 0.25
                        else:
                            ob2[r, sl] = o
                            yb2[r, sl] = t * sd
                return carry2
            lax.fori_loop(0, RCHUNK // L, _rows, 0)
            pltpu.sync_copy(zb2, acc.at[pl.ds(base, RCHUNK)])
            pltpu.sync_copy(ob2, out.at[c].at[pl.ds(base, RCHUNK)])
            if not last:
                pltpu.sync_copy(yb2, ynext.at[c].at[pl.ds(base, RCHUNK)])
            return carry
        lax.fori_loop(0, NRC, _post_chunk, 0)
        plsc.subcore_barrier()


@jax.jit
def _run(embs, srcp, dstp):
    mesh = plsc.VectorSubcoreMesh(core_axis_name="c", subcore_axis_name="s")
    f = pl.kernel(
        _body,
        out_type=[
            jax.ShapeDtypeStruct((NC, NPAD, H), jnp.float32),
            jax.ShapeDtypeStruct((NC, NPAD, H), jnp.float32),
            jax.ShapeDtypeStruct((NC, NPAD, H), jnp.float32),
        ],
        mesh=mesh,
        compiler_params=pltpu.CompilerParams(use_tc_tiling_on_sc=False),
        scratch_types=[
            pltpu.VMEM_SHARED((NPAD, H), jnp.float32),
            pltpu.VMEM_SHARED((NPAD,), jnp.float32),
            pltpu.VMEM((GROUP, CHUNK), jnp.int32),
            pltpu.VMEM((GROUP, CHUNK), jnp.int32),
            pltpu.VMEM((GROUP * CHUNK, H), jnp.float32),
            pltpu.VMEM((RCHUNK, H), jnp.float32),
            pltpu.VMEM((RCHUNK, H), jnp.float32),
            pltpu.VMEM((RCHUNK, H), jnp.float32),
            pltpu.VMEM((RCHUNK, H), jnp.float32),
            pltpu.VMEM((RCHUNK,), jnp.float32),
            pltpu.VMEM((CHUNK,), jnp.float32),
            pltpu.VMEM((RCHUNK,), jnp.float32),
            pltpu.SemaphoreType.DMA,
        ],
    )
    return f(embs, srcp, dstp)


def kernel(emb, edge_index):
    src = edge_index[0]
    dst = edge_index[1]
    pad = jnp.full((EPAD - E,), DUMMY, dtype=jnp.int32)
    srcp = jnp.concatenate([src, pad]).reshape(NCHUNKS, CHUNK)
    dstp = jnp.concatenate([dst, pad]).reshape(NCHUNKS, CHUNK)
    embp = jnp.pad(emb, ((0, NPAD - N), (0, 0)))
    embs = jnp.stack([embp[:, :H], embp[:, H:]])
    out, _, _ = _run(embs, srcp, dstp)
    return jnp.concatenate([out[0, :N], out[1, :N]], axis=1)


# speedup vs baseline: 7.8890x; 1.0624x over previous
import functools

import jax
import jax.numpy as jnp
from jax import lax
from jax.experimental import pallas as pl
from jax.experimental.pallas import tpu as pltpu
from jax.experimental.pallas import tpu_sc as plsc

N = 50000
D = 64
H = 32
E = 800000
NLAYERS = 3

NC, NS, L = 2, 16, 16

CHUNK = 128
GROUP = 2
CPT = 392
GPT = CPT // GROUP
EPAD = NS * CPT * CHUNK
NCHUNKS = EPAD // CHUNK

RPT = 3136
NPAD = NS * RPT
RCHUNK = 64
NRC = RPT // RCHUNK
DUMMY = N


def _rsqrt16(d):
    i = lax.bitcast_convert_type(d, jnp.int32)
    i = jnp.int32(0x5F3759DF) - lax.shift_right_arithmetic(i, 1)
    y = lax.bitcast_convert_type(i, jnp.float32)
    for _ in range(3):
        y = y * (1.5 - 0.5 * d * y * y)
    return y


def _body(embs, srcp, dstp, out, y0, y1, acc, deg,
          srcb0, srcb1, dstb0, dstb1, rows0, rows1,
          ab2, ob2, yb2, zb2, db, onesb, zb1, sem0, sem1):
    c = lax.axis_index("c")
    s = lax.axis_index("s")
    row_base = s * RPT
    chunk_base = s * CPT

    zero16 = jnp.zeros((L,), jnp.float32)
    one16 = jnp.ones((L,), jnp.float32)
    for r in range(RCHUNK):
        for h in range(H // L):
            zb2[r, pl.ds(h * L, L)] = zero16
    for v in range(RCHUNK // L):
        zb1[pl.ds(v * L, L)] = zero16
    for v in range(CHUNK // L):
        onesb[pl.ds(v * L, L)] = one16

    def _zero_slices(i, carry):
        base = row_base + i * RCHUNK
        pltpu.sync_copy(zb1, deg.at[pl.ds(base, RCHUNK)])
        pltpu.sync_copy(zb2, acc.at[pl.ds(base, RCHUNK)])
        return carry
    lax.fori_loop(0, NRC, _zero_slices, 0)
    plsc.subcore_barrier()

    def _deg_group(g, carry):
        cb = chunk_base + g * 4
        pltpu.sync_copy(dstp.at[pl.ds(cb, 2)], dstb0)
        pltpu.sync_copy(dstp.at[pl.ds(cb + 2, 2)], dstb1)
        descs = [pltpu.async_copy(onesb, deg.at[dstb0.at[j]], sem0, add=True)
                 for j in range(2)]
        descs += [pltpu.async_copy(onesb, deg.at[dstb1.at[j]], sem0, add=True)
                  for j in range(2)]
        for dsc in descs:
            dsc.wait()
        return carry
    lax.fori_loop(0, CPT // 4, _deg_group, 0)
    plsc.subcore_barrier()

    def _init_chunk(i, carry):
        base = row_base + i * RCHUNK
        pltpu.sync_copy(deg.at[pl.ds(base, RCHUNK)], db)
        pltpu.sync_copy(embs.at[c].at[pl.ds(base, RCHUNK)], ab2)

        def _rows(v, carry2):
            d = jnp.maximum(db[pl.ds(v * L, L)], 1.0)
            sdvec = _rsqrt16(d)
            for j in range(L):
                r = v * L + j
                sd = sdvec[j]
                for h in range(H // L):
                    e = ab2[r, pl.ds(h * L, L)]
                    yb2[r, pl.ds(h * L, L)] = e * sd
            return carry2
        lax.fori_loop(0, RCHUNK // L, _rows, 0)
        pltpu.sync_copy(ab2, out.at[c].at[pl.ds(base, RCHUNK)])
        pltpu.sync_copy(yb2, y0.at[c].at[pl.ds(base, RCHUNK)])
        return carry
    lax.fori_loop(0, NRC, _init_chunk, 0)
    plsc.subcore_barrier()

    ybufs = [y0, y1]
    for layer in range(NLAYERS):
        ycur = ybufs[layer % 2]
        ynext = ybufs[(layer + 1) % 2]

        def _fire(g, srcb, dstb, rows, sem):
            cb = chunk_base + g * GROUP
            pltpu.sync_copy(srcp.at[pl.ds(cb, GROUP)], srcb)
            pltpu.sync_copy(dstp.at[pl.ds(cb, GROUP)], dstb)
            for j in range(GROUP):
                pltpu.async_copy(ycur.at[c].at[srcb.at[j]],
                                 rows.at[pl.ds(j * CHUNK, CHUNK)], sem)

        def _drain_scatter(srcb, dstb, rows, sem):
            for j in range(GROUP):
                pltpu.make_async_copy(
                    ycur.at[c].at[srcb.at[j]],
                    rows.at[pl.ds(j * CHUNK, CHUNK)], sem).wait()
            for j in range(GROUP):
                pltpu.sync_copy(rows.at[pl.ds(j * CHUNK, CHUNK)],
                                acc.at[dstb.at[j]], add=True)

        _fire(0, srcb0, dstb0, rows0, sem0)

        def _agg_pair(i, carry):
            _fire(2 * i + 1, srcb1, dstb1, rows1, sem1)
            _drain_scatter(srcb0, dstb0, rows0, sem0)

            @pl.when(i < GPT // 2 - 1)
            def _():
                _fire(2 * i + 2, srcb0, dstb0, rows0, sem0)
            _drain_scatter(srcb1, dstb1, rows1, sem1)
            return carry
        lax.fori_loop(0, GPT // 2, _agg_pair, 0)
        plsc.subcore_barrier()

        last = layer == NLAYERS - 1

        def _post_chunk(i, carry):
            base = row_base + i * RCHUNK
            pltpu.sync_copy(deg.at[pl.ds(base, RCHUNK)], db)
            pltpu.sync_copy(acc.at[pl.ds(base, RCHUNK)], ab2)
            pltpu.sync_copy(out.at[c].at[pl.ds(base, RCHUNK)], ob2)

            def _rows(v, carry2):
                d = jnp.maximum(db[pl.ds(v * L, L)], 1.0)
                sdvec = _rsqrt16(d)
                for j in range(L):
                    r = v * L + j
                    sd = sdvec[j]
                    for h in range(H // L):
                        sl = pl.ds(h * L, L)
                        t = ab2[r, sl] * sd
                        o = ob2[r, sl] + t
                        if last:
                            ob2[r, sl] = o * 0.25
                        else:
                            ob2[r, sl] = o
                            yb2[r, sl] = t * sd
                return carry2
            lax.fori_loop(0, RCHUNK // L, _rows, 0)
            pltpu.sync_copy(zb2, acc.at[pl.ds(base, RCHUNK)])
            pltpu.sync_copy(ob2, out.at[c].at[pl.ds(base, RCHUNK)])
            if not last:
                pltpu.sync_copy(yb2, ynext.at[c].at[pl.ds(base, RCHUNK)])
            return carry
        lax.fori_loop(0, NRC, _post_chunk, 0)
        plsc.subcore_barrier()


@jax.jit
def _run(embs, srcp, dstp):
    mesh = plsc.VectorSubcoreMesh(core_axis_name="c", subcore_axis_name="s")
    f = pl.kernel(
        _body,
        out_type=[
            jax.ShapeDtypeStruct((NC, NPAD, H), jnp.float32),
            jax.ShapeDtypeStruct((NC, NPAD, H), jnp.float32),
            jax.ShapeDtypeStruct((NC, NPAD, H), jnp.float32),
        ],
        mesh=mesh,
        compiler_params=pltpu.CompilerParams(use_tc_tiling_on_sc=False),
        scratch_types=[
            pltpu.VMEM_SHARED((NPAD, H), jnp.float32),
            pltpu.VMEM_SHARED((NPAD,), jnp.float32),
            pltpu.VMEM((GROUP, CHUNK), jnp.int32),
            pltpu.VMEM((GROUP, CHUNK), jnp.int32),
            pltpu.VMEM((GROUP, CHUNK), jnp.int32),
            pltpu.VMEM((GROUP, CHUNK), jnp.int32),
            pltpu.VMEM((GROUP * CHUNK, H), jnp.float32),
            pltpu.VMEM((GROUP * CHUNK, H), jnp.float32),
            pltpu.VMEM((RCHUNK, H), jnp.float32),
            pltpu.VMEM((RCHUNK, H), jnp.float32),
            pltpu.VMEM((RCHUNK, H), jnp.float32),
            pltpu.VMEM((RCHUNK, H), jnp.float32),
            pltpu.VMEM((RCHUNK,), jnp.float32),
            pltpu.VMEM((CHUNK,), jnp.float32),
            pltpu.VMEM((RCHUNK,), jnp.float32),
            pltpu.SemaphoreType.DMA,
            pltpu.SemaphoreType.DMA,
        ],
    )
    return f(embs, srcp, dstp)


def kernel(emb, edge_index):
    src = edge_index[0]
    dst = edge_index[1]
    pad = jnp.full((EPAD - E,), DUMMY, dtype=jnp.int32)
    srcp = jnp.concatenate([src, pad]).reshape(NCHUNKS, CHUNK)
    dstp = jnp.concatenate([dst, pad]).reshape(NCHUNKS, CHUNK)
    embp = jnp.pad(emb, ((0, NPAD - N), (0, 0)))
    embs = jnp.stack([embp[:, :H], embp[:, H:]])
    out, _, _ = _run(embs, srcp, dstp)
    return jnp.concatenate([out[0, :N], out[1, :N]], axis=1)
